# baseline (speedup 1.0000x reference)
import jax
import jax.numpy as jnp
from jax import lax
from jax.experimental import pallas as pl
from jax.experimental.pallas import tpu as pltpu
from jax.experimental.pallas import tpu_sc as plsc

D = 64
ROWS = 4096 * 20
V = 1000000
TBK = 8192
NBQ = 31
Q = TBK * NBQ

_info = plsc.get_sparse_core_info()
NC, NS = _info.num_cores, _info.num_subcores
NW = NC * NS
B_PER_W = ROWS // NW
CHUNK = 640
N_CHUNKS = B_PER_W // CHUNK


def _bf16_hi(x):
    return x.astype(jnp.bfloat16).astype(jnp.float32)


def _tc_transpose(a_ref, b_ref, c_ref, d_ref, out_ref):
    ac = jnp.concatenate([a_ref[...], c_ref[...]], axis=0)
    bd = jnp.concatenate([b_ref[...], d_ref[...]], axis=0)
    hi = lax.bitcast_convert_type(_bf16_hi(jnp.swapaxes(ac, 0, 1)),
                                  jnp.uint32)
    lo = lax.bitcast_convert_type(_bf16_hi(jnp.swapaxes(bd, 0, 1)),
                                  jnp.uint32)
    out_ref[...] = lax.bitcast_convert_type(hi | (lo >> 16), jnp.float32)


def _pack_table(table_t):
    nb = V // TBK
    return pl.pallas_call(
        _tc_transpose,
        grid=(NBQ,),
        in_specs=[
            pl.BlockSpec((D, TBK), lambda c: (0, c)),
            pl.BlockSpec((D, TBK), lambda c: (0, c + NBQ)),
            pl.BlockSpec((D, TBK), lambda c: (0, c + 2 * NBQ)),
            pl.BlockSpec((D, TBK), lambda c: (0, jnp.minimum(c + 3 * NBQ, nb))),
        ],
        out_specs=pl.BlockSpec((TBK, 2 * D), lambda c: (c, 0)),
        out_shape=jax.ShapeDtypeStruct((Q, 2 * D), jnp.float32),
    )(table_t, table_t, table_t, table_t)


def _sc_flags(tok_hbm, flags_hbm, flag_out, idx_v, fval_v, sem):
    wid = lax.axis_index("s") * NC + lax.axis_index("c")
    base = wid * B_PER_W
    pltpu.sync_copy(tok_hbm.at[pl.ds(base, B_PER_W)], idx_v)
    pltpu.async_copy(flags_hbm.at[idx_v], fval_v, sem).wait()
    pltpu.sync_copy(fval_v, flag_out.at[pl.ds(base, B_PER_W)])


def _flags_call(tok, flags):
    mesh = plsc.VectorSubcoreMesh(core_axis_name="c", subcore_axis_name="s")
    return pl.kernel(
        _sc_flags,
        out_type=jax.ShapeDtypeStruct((ROWS,), jnp.int32),
        mesh=mesh,
        scratch_types=[
            pltpu.VMEM((B_PER_W,), jnp.int32),
            pltpu.VMEM((B_PER_W,), jnp.int32),
            pltpu.SemaphoreType.DMA,
        ],
        compiler_params=pltpu.CompilerParams(use_tc_tiling_on_sc=False),
    )(tok, flags)


def _sc_rows(tok_hbm, packed_hbm, emb_out, idx_v, rows_a, rows_b, sem_a, sem_b):
    wid = lax.axis_index("s") * NC + lax.axis_index("c")
    base = wid * B_PER_W
    half = wid // (NW // 2)
    p0 = base - half * (ROWS // 2)
    col = half * D
    pltpu.sync_copy(tok_hbm.at[pl.ds(base, B_PER_W)], idx_v)
    bufs = (rows_a, rows_b)
    sems = (sem_a, sem_b)

    def fire(c):
        return pltpu.async_copy(
            packed_hbm.at[idx_v.at[pl.ds(c * CHUNK, CHUNK)]], bufs[c % 2],
            sems[c % 2])

    cps = [None] * N_CHUNKS
    cps[0] = fire(0)
    for c in range(N_CHUNKS):
        if c + 1 < N_CHUNKS:
            cps[c + 1] = fire(c + 1)
        cps[c].wait()
        pltpu.sync_copy(bufs[c % 2],
                        emb_out.at[pl.ds(p0 + c * CHUNK, CHUNK),
                                   pl.ds(col, D)])


def _rows_call(tok_row, packed2):
    mesh = plsc.VectorSubcoreMesh(core_axis_name="c", subcore_axis_name="s")
    return pl.kernel(
        _sc_rows,
        out_type=jax.ShapeDtypeStruct((ROWS // 2, 2 * D), jnp.float32),
        mesh=mesh,
        scratch_types=[
            pltpu.VMEM((B_PER_W,), jnp.int32),
            pltpu.VMEM((CHUNK, D), jnp.float32),
            pltpu.VMEM((CHUNK, D), jnp.float32),
            pltpu.SemaphoreType.DMA,
            pltpu.SemaphoreType.DMA,
        ],
        compiler_params=pltpu.CompilerParams(use_tc_tiling_on_sc=False),
    )(tok_row, packed2)


def _tc_map(emb_ref, tokm2_ref, flag_ref, mask_ref, wt_ref, b_ref, out_ref):
    g = emb_ref[0]
    gt = lax.bitcast_convert_type(jnp.swapaxes(g, 0, 1), jnp.uint32)
    wt = wt_ref[...]
    b = b_ref[...]
    for h in range(2):
        grp = gt[h * D:(h + 1) * D]
        hi = tokm2_ref[h, 0] < Q
        bits = jnp.where(hi, grp & jnp.uint32(0xFFFF0000), grp << 16)
        et = lax.bitcast_convert_type(bits, jnp.float32)
        mapped = jnp.dot(wt, et, preferred_element_type=jnp.float32) + b
        sel = flag_ref[h, 0] != 0
        out_ref[h, 0] = jnp.where(sel, mapped, et) * mask_ref[h, 0]


def _map_call(emb3, tokm23, flag3, mask3, W_T, b_col, L, B):
    return pl.pallas_call(
        _tc_map,
        grid=(L // 2, 2),
        in_specs=[
            pl.BlockSpec((1, B // 2, 2 * D), lambda l, j: (l, j, 0)),
            pl.BlockSpec((2, 1, 1, B // 2), lambda l, j: (0, l, 0, j)),
            pl.BlockSpec((2, 1, 1, B // 2), lambda l, j: (0, l, 0, j)),
            pl.BlockSpec((2, 1, 1, B // 2), lambda l, j: (0, l, 0, j)),
            pl.BlockSpec((D, D), lambda l, j: (0, 0)),
            pl.BlockSpec((D, 1), lambda l, j: (0, 0)),
        ],
        out_specs=pl.BlockSpec((2, 1, D, B // 2), lambda l, j: (0, l, 0, j)),
        out_shape=jax.ShapeDtypeStruct((2, L // 2, D, B), jnp.float32),
    )(emb3, tokm23, flag3, mask3, W_T, b_col)


def kernel(token, mask, need_mapper, table, W_map, b_map):
    B, L = token.shape
    tok = token.T.reshape(-1).astype(jnp.int32)
    slot = tok // Q
    r = tok - slot * Q
    tok_row = 2 * r + (slot >> 1)
    tokm2 = tok - (slot >> 1) * (2 * Q)
    flags = need_mapper.astype(jnp.int32)
    packed = _pack_table(table.T)
    packed2 = packed.reshape(2 * Q, D)
    flagv = _flags_call(tok, flags)
    tok_row, flagv = lax.optimization_barrier((tok_row, flagv))
    emb = _rows_call(tok_row, packed2)
    emb3 = emb.reshape(L // 2, B, 2 * D)
    tokm23 = tokm2.reshape(2, L // 2, 1, B)
    flag3 = flagv.reshape(2, L // 2, 1, B)
    mask3 = mask.T.reshape(2, L // 2, 1, B)
    out_t = _map_call(emb3, tokm23, flag3, mask3, W_map.T,
                      b_map.reshape(D, 1), L, B)
    out_ldb = out_t.reshape(L, D, B)
    return out_ldb.transpose(2, 0, 1)


# speedup vs baseline: 1.0305x; 1.0305x over previous
import jax
import jax.numpy as jnp
from jax import lax
from jax.experimental import pallas as pl
from jax.experimental.pallas import tpu as pltpu
from jax.experimental.pallas import tpu_sc as plsc

D = 64
ROWS = 4096 * 20
V = 1000000
TBK = 8192
NBQ = 31
Q = TBK * NBQ

_info = plsc.get_sparse_core_info()
NC, NS = _info.num_cores, _info.num_subcores
NW = NC * NS
B_PER_W = ROWS // NW
CHUNK = 640
N_CHUNKS = B_PER_W // CHUNK


def _bf16_hi(x):
    return x.astype(jnp.bfloat16).astype(jnp.float32)


def _tc_transpose(a_ref, b_ref, c_ref, d_ref, out_ref):
    ac = jnp.concatenate([a_ref[...], c_ref[...]], axis=0)
    bd = jnp.concatenate([b_ref[...], d_ref[...]], axis=0)
    hi = lax.bitcast_convert_type(_bf16_hi(jnp.swapaxes(ac, 0, 1)),
                                  jnp.uint32)
    lo = lax.bitcast_convert_type(_bf16_hi(jnp.swapaxes(bd, 0, 1)),
                                  jnp.uint32)
    out_ref[...] = lax.bitcast_convert_type(hi | (lo >> 16), jnp.float32)


def _pack_table(table_t):
    nb = V // TBK
    return pl.pallas_call(
        _tc_transpose,
        grid=(NBQ,),
        in_specs=[
            pl.BlockSpec((D, TBK), lambda c: (0, c)),
            pl.BlockSpec((D, TBK), lambda c: (0, c + NBQ)),
            pl.BlockSpec((D, TBK), lambda c: (0, c + 2 * NBQ)),
            pl.BlockSpec((D, TBK), lambda c: (0, jnp.minimum(c + 3 * NBQ, nb))),
        ],
        out_specs=pl.BlockSpec((TBK, 2 * D), lambda c: (c, 0)),
        out_shape=jax.ShapeDtypeStruct((Q, 2 * D), jnp.float32),
    )(table_t, table_t, table_t, table_t)


def _sc_flags(tok_hbm, flags_hbm, flag_out, idx_v, fval_v, sem):
    wid = lax.axis_index("s") * NC + lax.axis_index("c")
    base = wid * B_PER_W
    pltpu.sync_copy(tok_hbm.at[pl.ds(base, B_PER_W)], idx_v)
    pltpu.async_copy(flags_hbm.at[idx_v], fval_v, sem).wait()
    pltpu.sync_copy(fval_v, flag_out.at[pl.ds(base, B_PER_W)])


def _flags_call(tok, flags):
    mesh = plsc.VectorSubcoreMesh(core_axis_name="c", subcore_axis_name="s")
    return pl.kernel(
        _sc_flags,
        out_type=jax.ShapeDtypeStruct((ROWS,), jnp.int32),
        mesh=mesh,
        scratch_types=[
            pltpu.VMEM((B_PER_W,), jnp.int32),
            pltpu.VMEM((B_PER_W,), jnp.int32),
            pltpu.SemaphoreType.DMA,
        ],
        compiler_params=pltpu.CompilerParams(use_tc_tiling_on_sc=False),
    )(tok, flags)


def _sc_rows(tok_hbm, packed_hbm, emb_out, idx_v, rows_a, rows_b, sem_a, sem_b):
    wid = lax.axis_index("s") * NC + lax.axis_index("c")
    base = wid * B_PER_W
    half = wid // (NW // 2)
    p0 = base - half * (ROWS // 2)
    col = half * D
    pltpu.sync_copy(tok_hbm.at[pl.ds(base, B_PER_W)], idx_v)
    bufs = (rows_a, rows_b)
    sems = (sem_a, sem_b)

    def fire(c):
        return pltpu.async_copy(
            packed_hbm.at[idx_v.at[pl.ds(c * CHUNK, CHUNK)]], bufs[c % 2],
            sems[c % 2])

    cps = [None] * N_CHUNKS
    cps[0] = fire(0)
    for c in range(N_CHUNKS):
        if c + 1 < N_CHUNKS:
            cps[c + 1] = fire(c + 1)
        cps[c].wait()
        pltpu.sync_copy(bufs[c % 2],
                        emb_out.at[pl.ds(p0 + c * CHUNK, CHUNK),
                                   pl.ds(col, D)])


def _rows_call(tok_row, packed2):
    mesh = plsc.VectorSubcoreMesh(core_axis_name="c", subcore_axis_name="s")
    return pl.kernel(
        _sc_rows,
        out_type=jax.ShapeDtypeStruct((ROWS // 2, 2 * D), jnp.float32),
        mesh=mesh,
        scratch_types=[
            pltpu.VMEM((B_PER_W,), jnp.int32),
            pltpu.VMEM((CHUNK, D), jnp.float32),
            pltpu.VMEM((CHUNK, D), jnp.float32),
            pltpu.SemaphoreType.DMA,
            pltpu.SemaphoreType.DMA,
        ],
        compiler_params=pltpu.CompilerParams(use_tc_tiling_on_sc=False),
    )(tok_row, packed2)


def _tc_map(emb_ref, tokm2_ref, flag_ref, mask_ref, wt_ref, b_ref, out_ref):
    g = emb_ref[0]
    gt = lax.bitcast_convert_type(jnp.swapaxes(g, 0, 1), jnp.uint32)
    wt = wt_ref[...]
    b = b_ref[...]
    for h in range(2):
        grp = gt[h * D:(h + 1) * D]
        hi = tokm2_ref[h, 0] < Q
        bits = jnp.where(hi, grp & jnp.uint32(0xFFFF0000), grp << 16)
        et = lax.bitcast_convert_type(bits, jnp.float32)
        mapped = jnp.dot(wt, et, preferred_element_type=jnp.float32) + b
        sel = flag_ref[h, 0] != 0
        out_ref[h, 0] = jnp.where(sel, mapped, et) * mask_ref[h, 0]


def _map_call(emb3, tokm23, flag3, mask3, W_T, b_col, L, B):
    return pl.pallas_call(
        _tc_map,
        grid=(L // 2,),
        in_specs=[
            pl.BlockSpec((1, B, 2 * D), lambda l: (l, 0, 0)),
            pl.BlockSpec((2, 1, 1, B), lambda l: (0, l, 0, 0)),
            pl.BlockSpec((2, 1, 1, B), lambda l: (0, l, 0, 0)),
            pl.BlockSpec((2, 1, 1, B), lambda l: (0, l, 0, 0)),
            pl.BlockSpec((D, D), lambda l: (0, 0)),
            pl.BlockSpec((D, 1), lambda l: (0, 0)),
        ],
        out_specs=pl.BlockSpec((2, 1, D, B), lambda l: (0, l, 0, 0)),
        out_shape=jax.ShapeDtypeStruct((2, L // 2, D, B), jnp.float32),
    )(emb3, tokm23, flag3, mask3, W_T, b_col)


def kernel(token, mask, need_mapper, table, W_map, b_map):
    B, L = token.shape
    tok = token.T.reshape(-1).astype(jnp.int32)
    slot = tok // Q
    r = tok - slot * Q
    tok_row = 2 * r + (slot >> 1)
    tokm2 = tok - (slot >> 1) * (2 * Q)
    flags = need_mapper.astype(jnp.int32)
    packed = _pack_table(table.T)
    packed2 = packed.reshape(2 * Q, D)
    flagv = _flags_call(tok, flags)
    tok_row, flagv = lax.optimization_barrier((tok_row, flagv))
    emb = _rows_call(tok_row, packed2)
    emb3 = emb.reshape(L // 2, B, 2 * D)
    tokm23 = tokm2.reshape(2, L // 2, 1, B)
    flag3 = flagv.reshape(2, L // 2, 1, B)
    mask3 = mask.T.reshape(2, L // 2, 1, B)
    out_t = _map_call(emb3, tokm23, flag3, mask3, W_map.T,
                      b_map.reshape(D, 1), L, B)
    out_ldb = out_t.reshape(L, D, B)
    return out_ldb.transpose(2, 0, 1)
